# Initial kernel scaffold; baseline (speedup 1.0000x reference)
import jax
import jax.numpy as jnp
from jax.experimental import pallas as pl


def kernel(x, edge_index, W0, al0, ar0, b0, W1, al1, ar1, b1, W2, al2, ar2, b2):
    raise NotImplementedError("write your pallas kernel here")


# speedup vs baseline: 18.9263x; 18.9263x over previous
import functools

import jax
import jax.numpy as jnp
from jax import lax
from jax.experimental import pallas as pl
from jax.experimental.pallas import tpu as pltpu
from jax.experimental.pallas import tpu_sc as plsc

N = 10000
E = 320000
H = 4
D = 32
HD = H * D
C = 40
PAD = HD + 16
NC = 2
NS = 16
NW = NC * NS
EW = E // NW
CH = 40
NCH = EW // CH
RPW = N // NS


def _mesh():
    return plsc.VectorSubcoreMesh(
        core_axis_name="c", subcore_axis_name="s",
        num_cores=NC, num_subcores=NS)


def _tc_head_body(h_ref, wp_ref, wal_ref, war_ref, feat_ref, el_ref, er_ref):
    h = h_ref[...]
    feat_ref[...] = jnp.dot(h, wp_ref[...], preferred_element_type=jnp.float32)
    el_ref[...] = jnp.dot(h, wal_ref[...], preferred_element_type=jnp.float32)
    er_ref[...] = jnp.dot(h, war_ref[...], preferred_element_type=jnp.float32)


def _tc_head(h, wp, wal, war):
    return pl.pallas_call(
        _tc_head_body,
        out_shape=(
            jax.ShapeDtypeStruct((N, HD), jnp.float32),
            jax.ShapeDtypeStruct((N, 4), jnp.float32),
            jax.ShapeDtypeStruct((N, 4), jnp.float32),
        ),
    )(h, wp, wal, war)


def _tc_mid_body(acc_ref, expm_ref, b_ref, wp_ref, wal_ref, war_ref,
                 feat_ref, el_ref, er_ref):
    hs = acc_ref[0] + acc_ref[1]
    den = jnp.dot(hs, expm_ref[...], preferred_element_type=jnp.float32)
    h = hs[:, :HD] / (den + 1e-9) + b_ref[...]
    feat_ref[...] = jnp.dot(h, wp_ref[...], preferred_element_type=jnp.float32)
    el_ref[...] = jnp.dot(h, wal_ref[...], preferred_element_type=jnp.float32)
    er_ref[...] = jnp.dot(h, war_ref[...], preferred_element_type=jnp.float32)


def _tc_mid(acc, expm, b, wp, wal, war):
    return pl.pallas_call(
        _tc_mid_body,
        out_shape=(
            jax.ShapeDtypeStruct((N, HD), jnp.float32),
            jax.ShapeDtypeStruct((N, 4), jnp.float32),
            jax.ShapeDtypeStruct((N, 4), jnp.float32),
        ),
    )(acc, expm, b, wp, wal, war)


def _tc_final_body(acc_ref, selm_ref, b_ref, out_ref):
    hs = acc_ref[0] + acc_ref[1]
    den = jnp.dot(hs, selm_ref[...], preferred_element_type=jnp.float32)
    out_ref[...] = hs[:, :C] / (den + 1e-9) + b_ref[...]


def _tc_final(acc, selm, b):
    return pl.pallas_call(
        _tc_final_body,
        out_shape=jax.ShapeDtypeStruct((N, C), jnp.float32),
    )(acc, selm, b)


def _edge_body(nheads, src_hbm, dst_hbm, feat_hbm, elr_hbm, acc_hbm,
               src_w, dst_w, elb, erb, eep, rowsF, rows, sem, acc_sh):
    cid = lax.axis_index("c")
    sid = lax.axis_index("s")
    wid = sid * NC + cid
    lanes = jnp.arange(16, dtype=jnp.int32)
    zero16 = jnp.zeros((16,), jnp.float32)

    def zrow(e, carry):
        for v in range(PAD // 16):
            rows[e, pl.ds(16 * v, 16)] = zero16
        eep[e, :] = zero16
        return carry
    lax.fori_loop(0, CH, zrow, 0)
    rbase = sid * RPW
    for i in range(0, RPW, CH):
        r = min(CH, RPW - i)
        pltpu.sync_copy(rows.at[pl.ds(0, r)], acc_sh.at[pl.ds(rbase + i, r)])
    plsc.subcore_barrier()

    ebase = wid * EW

    def chunk(c, carry):
        pltpu.sync_copy(src_hbm.at[pl.ds(ebase + c * CH, CH)], src_w)
        pltpu.sync_copy(dst_hbm.at[pl.ds(ebase + c * CH, CH)], dst_w)
        cp_el = pltpu.async_copy(elr_hbm.at[src_w], elb, sem)
        cp_er = pltpu.async_copy(elr_hbm.at[dst_w], erb, sem)
        cp_ft = pltpu.async_copy(feat_hbm.at[src_w], rowsF, sem)
        cp_el.wait()
        cp_er.wait()
        cp_ft.wait()

        if nheads == 4:
            def eeiter(j, cy):
                erow = 4 * j + (lanes >> 2)
                ecol = lanes & 3
                elv = plsc.load_gather(elb, [erow, ecol])
                erv = plsc.load_gather(erb, [erow, ecol + 4])
                s = elv + erv
                ee = jnp.exp(jnp.where(s > 0, s, 0.2 * s))
                plsc.store_scatter(eep, [erow, ecol], ee)
                return cy
            lax.fori_loop(0, CH // 4, eeiter, 0)
        else:
            def eeiter(j, cy):
                erow = 16 * j + lanes
                msk = erow < CH
                erow_c = jnp.minimum(erow, CH - 1)
                zc = jnp.zeros((16,), jnp.int32)
                elv = plsc.load_gather(elb, [erow_c, zc], mask=msk)
                erv = plsc.load_gather(erb, [erow_c, zc + 4], mask=msk)
                s = elv + erv
                ee = jnp.exp(jnp.where(s > 0, s, 0.2 * s))
                plsc.store_scatter(eep, [erow_c, zc], ee, mask=msk)
                return cy
            lax.fori_loop(0, (CH + 15) // 16, eeiter, 0)

        def mul(e, cy):
            eev = eep[e, :]
            if nheads == 4:
                es = (eev[0], eev[0], eev[1], eev[1],
                      eev[2], eev[2], eev[3], eev[3])
            else:
                e0 = eev[0]
                es = (e0,) * 8
            for v in range(HD // 16):
                rows[e, pl.ds(16 * v, 16)] = rowsF[e, pl.ds(16 * v, 16)] * es[v]
            rows[e, pl.ds(HD, 16)] = eev
            return cy
        lax.fori_loop(0, CH, mul, 0)

        pltpu.sync_copy(rows, acc_sh.at[dst_w], add=True)
        return carry
    lax.fori_loop(0, NCH, chunk, 0)

    plsc.subcore_barrier()
    obase = (cid * N + rbase) * PAD

    def wout(b, carry):
        rb = rbase + b * 25
        ob = obase + b * 25 * PAD
        cps = [pltpu.async_copy(acc_sh.at[rb + i],
                                acc_hbm.at[pl.ds(ob + i * PAD, PAD)], sem)
               for i in range(25)]
        for cp in cps:
            cp.wait()
        return carry
    lax.fori_loop(0, RPW // 25, wout, 0)


def _edge_pass(nheads, src1, dst1, feat1, elr1):
    f = pl.kernel(
        functools.partial(_edge_body, nheads),
        out_type=jax.ShapeDtypeStruct((NC * N * PAD,), jnp.float32),
        mesh=_mesh(),
        compiler_params=pltpu.CompilerParams(
            use_tc_tiling_on_sc=False, needs_layout_passes=False),
        scratch_types=[
            pltpu.VMEM((CH,), jnp.int32),
            pltpu.VMEM((CH,), jnp.int32),
            pltpu.VMEM((CH, HD), jnp.float32),
            pltpu.VMEM((CH, HD), jnp.float32),
            pltpu.VMEM((CH, 16), jnp.float32),
            pltpu.VMEM((CH, HD), jnp.float32),
            pltpu.VMEM((CH, PAD), jnp.float32),
            pltpu.SemaphoreType.DMA,
            pltpu.VMEM_SHARED((N, PAD), jnp.float32),
        ],
    )
    return f(src1, dst1, feat1, elr1)


def _expand(a):
    nh, d = a.shape
    return (a[:, :, None] * jnp.eye(nh, dtype=jnp.float32)[:, None, :]).reshape(
        nh * d, nh)


def kernel(x, edge_index, W0, al0, ar0, b0, W1, al1, ar1, b1, W2, al2, ar2, b2):
    src1 = edge_index[0]
    dst1 = edge_index[1]

    W2p = jnp.concatenate([W2, jnp.zeros((HD, HD - C), jnp.float32)], axis=1)
    WAL0 = W0 @ _expand(al0)
    WAR0 = W0 @ _expand(ar0)
    WAL1 = W1 @ _expand(al1)
    WAR1 = W1 @ _expand(ar1)
    z3 = jnp.zeros((HD, 3), jnp.float32)
    WAL2 = jnp.concatenate([W2 @ al2[0][:, None], z3], axis=1)
    WAR2 = jnp.concatenate([W2 @ ar2[0][:, None], z3], axis=1)
    expm = jnp.zeros((PAD, HD), jnp.float32)
    rid = jnp.arange(HD)
    expm = expm.at[HD + rid // D, rid].set(1.0)
    selm = jnp.zeros((PAD, C), jnp.float32).at[HD, :].set(1.0)
    b0r = b0.reshape(1, HD)
    b1r = b1.reshape(1, HD)
    b2r = b2.reshape(1, C)

    def pack(feat, el, er):
        elr = jnp.concatenate(
            [el, er, jnp.zeros((N, HD - 8), jnp.float32)], axis=1)
        return feat, elr

    feat0, el0, er0 = _tc_head(x, W0, WAL0, WAR0)
    f1d, elr1d = pack(feat0, el0, er0)
    acc0 = _edge_pass(4, src1, dst1, f1d, elr1d).reshape(NC, N, PAD)
    feat1, el1, er1 = _tc_mid(acc0, expm, b0r, W1, WAL1, WAR1)
    f1d, elr1d = pack(feat1, el1, er1)
    acc1 = _edge_pass(4, src1, dst1, f1d, elr1d).reshape(NC, N, PAD)
    feat2, el2, er2 = _tc_mid(acc1, expm, b1r, W2p, WAL2, WAR2)
    f1d, elr1d = pack(feat2, el2, er2)
    acc2 = _edge_pass(1, src1, dst1, f1d, elr1d).reshape(NC, N, PAD)
    return _tc_final(acc2, selm, b2r)
